# baseline (speedup 1.0000x reference)
import functools

import jax
import jax.numpy as jnp
from jax import lax
from jax.experimental import pallas as pl
from jax.experimental.pallas import tpu as pltpu
from jax.experimental.pallas import tpu_sc as plsc

NUM_ENT = 1000000
NUM_REL = 1000
EMB_DIM = 64
BATCH = 16384

NC = 2
NS = 16
NW = NC * NS
B_PER_W = BATCH // NW
CHUNK = 64
NCHUNK = B_PER_W // CHUNK
L = 16
NG = EMB_DIM // L
SLOTS = 7 * CHUNK


def _mdist_kernel(e3, r3, i0, i1, i2, i3, i4, i5, i6,
                  out_hbm, idx_v, rows_v, out_v, sem0, sem1):
    wid = lax.axis_index("s") * NC + lax.axis_index("c")
    base = wid * B_PER_W

    for k, ih in enumerate((i0, i1, i2, i3, i4, i5, i6)):
        pltpu.sync_copy(ih.at[pl.ds(base, B_PER_W)],
                        idx_v.at[pl.ds(k * B_PER_W, B_PER_W)])

    iota = lax.broadcasted_iota(jnp.int32, (L,), 0)

    def fire(ch, buf, sem):
        for k in range(7):
            tbl = r3 if k == 0 else e3

            def fbody(v, carry, k=k, tbl=tbl):
                ivec = idx_v[pl.ds(k * B_PER_W + ch * CHUNK + v * L, L)]
                svec = ivec >> 3
                ubvec = ivec & 7
                for j2 in range(L):
                    slot = k * CHUNK + v * L + j2
                    pltpu.async_copy(
                        tbl.at[svec[j2], ubvec[j2]],
                        rows_v.at[buf, slot // 8, slot % 8], sem)
                return carry

            lax.fori_loop(0, CHUNK // L, fbody, 0)

    def waitall(buf, sem):
        pltpu.make_async_copy(
            e3.at[pl.ds(0, SLOTS // 8)], rows_v.at[buf], sem).wait()

    def compute(ch, buf):
        bufv = jnp.full((L,), buf, jnp.int32)
        for g in range(CHUNK // L):
            b16 = g * L + iota
            sub16 = b16 & 7
            row16 = b16 >> 3
            c1 = [row16 + 8 * k for k in range(7)]

            def dbody(d, acc):
                lane = jnp.full((L,), d, jnp.int32)
                p = None
                for k in range(7):
                    x = plsc.load_gather(rows_v, [bufv, c1[k], sub16, lane])
                    p = x if p is None else p * x
                return acc + p

            acc = lax.fori_loop(0, EMB_DIM, dbody,
                                jnp.zeros((L,), jnp.float32), unroll=4)
            out_v[pl.ds(ch * CHUNK + g * L, L)] = acc

    fire(0, 0, sem0)

    def pair(p, carry):
        ch0 = 2 * p
        fire(ch0 + 1, 1, sem1)
        waitall(0, sem0)
        compute(ch0, 0)
        fire(ch0 + 2, 0, sem0)
        waitall(1, sem1)
        compute(ch0 + 1, 1)
        return carry

    lax.fori_loop(0, NCHUNK // 2 - 1, pair, 0)
    fire(NCHUNK - 1, 1, sem1)
    waitall(0, sem0)
    compute(NCHUNK - 2, 0)
    waitall(1, sem1)
    compute(NCHUNK - 1, 1)

    pltpu.sync_copy(out_v, out_hbm.at[pl.ds(base, B_PER_W)])


@jax.jit
def _mdist(e3, r3, i0, i1, i2, i3, i4, i5, i6):
    mesh = plsc.VectorSubcoreMesh(core_axis_name="c", subcore_axis_name="s")
    run = functools.partial(
        pl.kernel,
        mesh=mesh,
        compiler_params=pltpu.CompilerParams(needs_layout_passes=False),
        out_type=jax.ShapeDtypeStruct((BATCH,), jnp.float32),
        scratch_types=[
            pltpu.VMEM((7 * B_PER_W,), jnp.int32),
            pltpu.VMEM((2, SLOTS // 8, 8, EMB_DIM), jnp.float32),
            pltpu.VMEM((B_PER_W,), jnp.float32),
            pltpu.SemaphoreType.DMA,
            pltpu.SemaphoreType.DMA,
        ],
    )(_mdist_kernel)
    return run(e3, r3, i0, i1, i2, i3, i4, i5, i6)


def kernel(r_idx, e1_idx, e2_idx, e3_idx, e4_idx, e5_idx, e6_idx,
           E_weight, R_weight):
    e3 = E_weight.reshape(NUM_ENT // 8, 8, EMB_DIM)
    r3 = R_weight.reshape(NUM_REL // 8, 8, EMB_DIM)
    return _mdist(e3, r3,
                  r_idx.astype(jnp.int32), e1_idx.astype(jnp.int32),
                  e2_idx.astype(jnp.int32), e3_idx.astype(jnp.int32),
                  e4_idx.astype(jnp.int32), e5_idx.astype(jnp.int32),
                  e6_idx.astype(jnp.int32))


# speedup vs baseline: 1.1051x; 1.1051x over previous
import functools

import jax
import jax.numpy as jnp
from jax import lax
from jax.experimental import pallas as pl
from jax.experimental.pallas import tpu as pltpu
from jax.experimental.pallas import tpu_sc as plsc

NUM_ENT = 1000000
NUM_REL = 1000
EMB_DIM = 64
BATCH = 16384

NC = 2
NS = 16
NW = NC * NS
B_PER_W = BATCH // NW
CHUNK = 32
NCHUNK = B_PER_W // CHUNK
L = 16
NG = EMB_DIM // L
SLOTS = 7 * CHUNK


def _mdist_kernel(e3, r3, i0, i1, i2, i3, i4, i5, i6,
                  out_hbm, idx_v, rows_v, out_v, sem0, sem1):
    wid = lax.axis_index("s") * NC + lax.axis_index("c")
    base = wid * B_PER_W

    for k, ih in enumerate((i0, i1, i2, i3, i4, i5, i6)):
        pltpu.sync_copy(ih.at[pl.ds(base, B_PER_W)],
                        idx_v.at[pl.ds(k * B_PER_W, B_PER_W)])

    iota = lax.broadcasted_iota(jnp.int32, (L,), 0)

    def fire(ch, buf, sem):
        for k in range(7):
            tbl = r3 if k == 0 else e3

            def fbody(v, carry, k=k, tbl=tbl):
                ivec = idx_v[pl.ds(k * B_PER_W + ch * CHUNK + v * L, L)]
                svec = ivec >> 3
                ubvec = ivec & 7
                for j2 in range(L):
                    slot = k * CHUNK + v * L + j2
                    pltpu.async_copy(
                        tbl.at[svec[j2], ubvec[j2]],
                        rows_v.at[buf, slot // 8, slot % 8], sem)
                return carry

            lax.fori_loop(0, CHUNK // L, fbody, 0)

    def waitall(buf, sem):
        pltpu.make_async_copy(
            e3.at[pl.ds(0, SLOTS // 8)], rows_v.at[buf], sem).wait()

    def compute(ch, buf):
        for g in range(CHUNK // L):

            def rbody(j, sums):
                b = g * L + j
                acc = None
                for gg in range(NG):
                    p = None
                    for k in range(7):
                        slot = k * CHUNK + b
                        x = rows_v[buf, slot // 8, slot % 8,
                                   pl.ds(gg * L, L)]
                        p = x if p is None else p * x
                    acc = p if acc is None else acc + p
                s = jnp.sum(acc)
                return jnp.where(iota == j, s, sums)

            sums = lax.fori_loop(0, L, rbody, jnp.zeros((L,), jnp.float32))
            out_v[pl.ds(ch * CHUNK + g * L, L)] = sums

    def step(ch_next, ch_cur, buf_fire, sem_fire, buf_cur):
        for g in range(CHUNK // L):
            for k in range(7):
                tbl = r3 if k == 0 else e3
                ivec = idx_v[pl.ds(k * B_PER_W + ch_next * CHUNK + g * L, L)]
                svec = ivec >> 3
                ubvec = ivec & 7
                for j2 in range(L):
                    slot = k * CHUNK + g * L + j2
                    pltpu.async_copy(
                        tbl.at[svec[j2], ubvec[j2]],
                        rows_v.at[buf_fire, slot // 8, slot % 8], sem_fire)
            sums = jnp.zeros((L,), jnp.float32)
            for j in range(L):
                b = g * L + j
                acc = None
                for gg in range(NG):
                    p = None
                    for k in range(7):
                        slot = k * CHUNK + b
                        x = rows_v[buf_cur, slot // 8, slot % 8,
                                   pl.ds(gg * L, L)]
                        p = x if p is None else p * x
                    acc = p if acc is None else acc + p
                s = jnp.sum(acc)
                sums = jnp.where(iota == j, s, sums)
            out_v[pl.ds(ch_cur * CHUNK + g * L, L)] = sums

    fire(0, 0, sem0)

    def pair(p, carry):
        ch0 = 2 * p
        waitall(0, sem0)
        step(ch0 + 1, ch0, 1, sem1, 0)
        waitall(1, sem1)
        step(ch0 + 2, ch0 + 1, 0, sem0, 1)
        return carry

    lax.fori_loop(0, NCHUNK // 2 - 1, pair, 0)
    fire(NCHUNK - 1, 1, sem1)
    waitall(0, sem0)
    compute(NCHUNK - 2, 0)
    waitall(1, sem1)
    compute(NCHUNK - 1, 1)

    pltpu.sync_copy(out_v, out_hbm.at[pl.ds(base, B_PER_W)])


@jax.jit
def _mdist(e3, r3, i0, i1, i2, i3, i4, i5, i6):
    mesh = plsc.VectorSubcoreMesh(core_axis_name="c", subcore_axis_name="s")
    run = functools.partial(
        pl.kernel,
        mesh=mesh,
        compiler_params=pltpu.CompilerParams(needs_layout_passes=False),
        out_type=jax.ShapeDtypeStruct((BATCH,), jnp.float32),
        scratch_types=[
            pltpu.VMEM((7 * B_PER_W,), jnp.int32),
            pltpu.VMEM((2, SLOTS // 8, 8, EMB_DIM), jnp.float32),
            pltpu.VMEM((B_PER_W,), jnp.float32),
            pltpu.SemaphoreType.DMA,
            pltpu.SemaphoreType.DMA,
        ],
    )(_mdist_kernel)
    return run(e3, r3, i0, i1, i2, i3, i4, i5, i6)


def kernel(r_idx, e1_idx, e2_idx, e3_idx, e4_idx, e5_idx, e6_idx,
           E_weight, R_weight):
    e3 = E_weight.reshape(NUM_ENT // 8, 8, EMB_DIM)
    r3 = R_weight.reshape(NUM_REL // 8, 8, EMB_DIM)
    return _mdist(e3, r3,
                  r_idx.astype(jnp.int32), e1_idx.astype(jnp.int32),
                  e2_idx.astype(jnp.int32), e3_idx.astype(jnp.int32),
                  e4_idx.astype(jnp.int32), e5_idx.astype(jnp.int32),
                  e6_idx.astype(jnp.int32))


# speedup vs baseline: 1.3101x; 1.1856x over previous
import functools

import jax
import jax.numpy as jnp
from jax import lax
from jax.experimental import pallas as pl
from jax.experimental.pallas import tpu as pltpu
from jax.experimental.pallas import tpu_sc as plsc

NUM_ENT = 1000000
NUM_REL = 1000
EMB_DIM = 64
BATCH = 16384

NC = 2
NS = 16
NW = NC * NS
B_PER_W = BATCH // NW
CHUNK = 64
NCHUNK = B_PER_W // CHUNK
L = 16
NG = EMB_DIM // L
SLOTS = 7 * CHUNK


def _mdist_kernel(e3, r3, i0, i1, i2, i3, i4, i5, i6,
                  out_hbm, idx_v, rows_v, out_v, sem0, sem1):
    wid = lax.axis_index("s") * NC + lax.axis_index("c")
    base = wid * B_PER_W

    for k, ih in enumerate((i0, i1, i2, i3, i4, i5, i6)):
        pltpu.sync_copy(ih.at[pl.ds(base, B_PER_W)],
                        idx_v.at[pl.ds(k * B_PER_W, B_PER_W)])

    iota = lax.broadcasted_iota(jnp.int32, (L,), 0)

    def fire(ch, buf, sem):
        for k in range(7):
            tbl = r3 if k == 0 else e3

            def fbody(v, carry, k=k, tbl=tbl):
                ivec = idx_v[pl.ds(k * B_PER_W + ch * CHUNK + v * L, L)]
                svec = ivec >> 3
                ubvec = ivec & 7
                for j2 in range(L):
                    slot = k * CHUNK + v * L + j2
                    pltpu.async_copy(
                        tbl.at[svec[j2], ubvec[j2]],
                        rows_v.at[buf, slot // 8, slot % 8], sem)
                return carry

            lax.fori_loop(0, CHUNK // L, fbody, 0)

    def waitall(buf, sem):
        pltpu.make_async_copy(
            e3.at[pl.ds(0, SLOTS // 8)], rows_v.at[buf], sem).wait()

    def compute(ch, buf):
        for g in range(CHUNK // L):

            def rbody(j, sums):
                b = g * L + j
                acc = None
                for gg in range(NG):
                    p = None
                    for k in range(7):
                        slot = k * CHUNK + b
                        x = rows_v[buf, slot // 8, slot % 8,
                                   pl.ds(gg * L, L)]
                        p = x if p is None else p * x
                    acc = p if acc is None else acc + p
                s = jnp.sum(acc)
                return jnp.where(iota == j, s, sums)

            sums = lax.fori_loop(0, L, rbody, jnp.zeros((L,), jnp.float32),
                                 unroll=4)
            out_v[pl.ds(ch * CHUNK + g * L, L)] = sums

    fire(0, 0, sem0)

    def pair(p, carry):
        ch0 = 2 * p
        fire(ch0 + 1, 1, sem1)
        waitall(0, sem0)
        compute(ch0, 0)
        fire(ch0 + 2, 0, sem0)
        waitall(1, sem1)
        compute(ch0 + 1, 1)
        return carry

    lax.fori_loop(0, NCHUNK // 2 - 1, pair, 0)
    fire(NCHUNK - 1, 1, sem1)
    waitall(0, sem0)
    compute(NCHUNK - 2, 0)
    waitall(1, sem1)
    compute(NCHUNK - 1, 1)

    pltpu.sync_copy(out_v, out_hbm.at[pl.ds(base, B_PER_W)])


@jax.jit
def _mdist(e3, r3, i0, i1, i2, i3, i4, i5, i6):
    mesh = plsc.VectorSubcoreMesh(core_axis_name="c", subcore_axis_name="s")
    run = functools.partial(
        pl.kernel,
        mesh=mesh,
        compiler_params=pltpu.CompilerParams(needs_layout_passes=False),
        out_type=jax.ShapeDtypeStruct((BATCH,), jnp.float32),
        scratch_types=[
            pltpu.VMEM((7 * B_PER_W,), jnp.int32),
            pltpu.VMEM((2, SLOTS // 8, 8, EMB_DIM), jnp.float32),
            pltpu.VMEM((B_PER_W,), jnp.float32),
            pltpu.SemaphoreType.DMA,
            pltpu.SemaphoreType.DMA,
        ],
    )(_mdist_kernel)
    return run(e3, r3, i0, i1, i2, i3, i4, i5, i6)


def kernel(r_idx, e1_idx, e2_idx, e3_idx, e4_idx, e5_idx, e6_idx,
           E_weight, R_weight):
    e3 = E_weight.reshape(NUM_ENT // 8, 8, EMB_DIM)
    r3 = R_weight.reshape(NUM_REL // 8, 8, EMB_DIM)
    return _mdist(e3, r3,
                  r_idx.astype(jnp.int32), e1_idx.astype(jnp.int32),
                  e2_idx.astype(jnp.int32), e3_idx.astype(jnp.int32),
                  e4_idx.astype(jnp.int32), e5_idx.astype(jnp.int32),
                  e6_idx.astype(jnp.int32))


# speedup vs baseline: 1.3471x; 1.0282x over previous
import functools

import jax
import jax.numpy as jnp
from jax import lax
from jax.experimental import pallas as pl
from jax.experimental.pallas import tpu as pltpu
from jax.experimental.pallas import tpu_sc as plsc

NUM_ENT = 1000000
NUM_REL = 1000
EMB_DIM = 64
BATCH = 16384

NC = 2
NS = 16
NW = NC * NS
B_PER_W = BATCH // NW
CHUNK = 64
NCHUNK = B_PER_W // CHUNK
L = 16
NG = EMB_DIM // L
SLOTS = 7 * CHUNK


def _mdist_kernel(e3, r3, i0, i1, i2, i3, i4, i5, i6,
                  out_hbm, idx_v, rows_v, out_v, sem0, sem1):
    wid = lax.axis_index("s") * NC + lax.axis_index("c")
    base = wid * B_PER_W

    for k, ih in enumerate((i0, i1, i2, i3, i4, i5, i6)):
        pltpu.sync_copy(ih.at[pl.ds(base, B_PER_W)],
                        idx_v.at[pl.ds(k * B_PER_W, B_PER_W)])

    iota = lax.broadcasted_iota(jnp.int32, (L,), 0)

    def fire(ch, buf, sem):
        for k in range(7):
            tbl = r3 if k == 0 else e3

            def fbody(v, carry, k=k, tbl=tbl):
                ivec = idx_v[pl.ds(k * B_PER_W + ch * CHUNK + v * L, L)]
                svec = ivec >> 3
                ubvec = ivec & 7
                for j2 in range(L):
                    slot = k * CHUNK + v * L + j2
                    pltpu.async_copy(
                        tbl.at[svec[j2], ubvec[j2]],
                        rows_v.at[buf, slot // 8, slot % 8], sem)
                return carry

            lax.fori_loop(0, CHUNK // L, fbody, 0)

    def waitall(buf, sem):
        pltpu.make_async_copy(
            e3.at[pl.ds(0, SLOTS // 8)], rows_v.at[buf], sem).wait()

    def compute(ch, buf):
        for g in range(CHUNK // L):

            def rbody(j, sums):
                b = g * L + j
                acc = None
                for gg in range(NG):
                    p = None
                    for k in range(7):
                        slot = k * CHUNK + b
                        x = rows_v[buf, slot // 8, slot % 8,
                                   pl.ds(gg * L, L)]
                        p = x if p is None else p * x
                    acc = p if acc is None else acc + p
                s = jnp.sum(acc)
                return jnp.where(iota == j, s, sums)

            sums = lax.fori_loop(0, L, rbody, jnp.zeros((L,), jnp.float32))
            out_v[pl.ds(ch * CHUNK + g * L, L)] = sums

    fire(0, 0, sem0)

    def pair(p, carry):
        ch0 = 2 * p
        fire(ch0 + 1, 1, sem1)
        waitall(0, sem0)
        compute(ch0, 0)
        fire(ch0 + 2, 0, sem0)
        waitall(1, sem1)
        compute(ch0 + 1, 1)
        return carry

    lax.fori_loop(0, NCHUNK // 2 - 1, pair, 0)
    fire(NCHUNK - 1, 1, sem1)
    waitall(0, sem0)
    compute(NCHUNK - 2, 0)
    waitall(1, sem1)
    compute(NCHUNK - 1, 1)

    pltpu.sync_copy(out_v, out_hbm.at[pl.ds(base, B_PER_W)])


@jax.jit
def _mdist(e3, r3, i0, i1, i2, i3, i4, i5, i6):
    mesh = plsc.VectorSubcoreMesh(core_axis_name="c", subcore_axis_name="s")
    run = functools.partial(
        pl.kernel,
        mesh=mesh,
        compiler_params=pltpu.CompilerParams(needs_layout_passes=False),
        out_type=jax.ShapeDtypeStruct((BATCH,), jnp.float32),
        scratch_types=[
            pltpu.VMEM((7 * B_PER_W,), jnp.int32),
            pltpu.VMEM((2, SLOTS // 8, 8, EMB_DIM), jnp.float32),
            pltpu.VMEM((B_PER_W,), jnp.float32),
            pltpu.SemaphoreType.DMA,
            pltpu.SemaphoreType.DMA,
        ],
    )(_mdist_kernel)
    return run(e3, r3, i0, i1, i2, i3, i4, i5, i6)


def kernel(r_idx, e1_idx, e2_idx, e3_idx, e4_idx, e5_idx, e6_idx,
           E_weight, R_weight):
    e3 = E_weight.reshape(NUM_ENT // 8, 8, EMB_DIM)
    r3 = R_weight.reshape(NUM_REL // 8, 8, EMB_DIM)
    return _mdist(e3, r3,
                  r_idx.astype(jnp.int32), e1_idx.astype(jnp.int32),
                  e2_idx.astype(jnp.int32), e3_idx.astype(jnp.int32),
                  e4_idx.astype(jnp.int32), e5_idx.astype(jnp.int32),
                  e6_idx.astype(jnp.int32))
